# baseline (speedup 1.0000x reference)
import jax
import jax.numpy as jnp
from jax.experimental import pallas as pl
from jax.experimental.pallas import tpu as pltpu


def _copy_kernel(x_ref, o_ref):
    o_ref[...] = x_ref[...]


def kernel(x, W1, b1, W2, b2):
    del W1, b1, W2, b2
    xr = x.reshape(3, 32768)
    out = pl.pallas_call(
        _copy_kernel,
        grid=(2,),
        in_specs=[pl.BlockSpec((3, 16384), lambda i: (0, i))],
        out_specs=pl.BlockSpec((3, 16384), lambda i: (0, i)),
        out_shape=jax.ShapeDtypeStruct(xr.shape, xr.dtype),
        compiler_params=pltpu.CompilerParams(
            dimension_semantics=("parallel",),
            skip_device_barrier=True,
            disable_bounds_checks=True,
            disable_semaphore_checks=True,
        ),
    )(xr)
    return out.reshape(x.shape)


# speedup vs baseline: 1.0262x; 1.0262x over previous
import jax
import jax.numpy as jnp
from jax.experimental import pallas as pl
from jax.experimental.pallas import tpu as pltpu


def _copy_kernel(x_ref, o_ref):
    o_ref[...] = x_ref[...]


def kernel(x, W1, b1, W2, b2):
    del W1, b1, W2, b2
    xr = x.reshape(3, 32768)
    out = pl.pallas_call(
        _copy_kernel,
        grid=(2,),
        in_specs=[pl.BlockSpec((3, 16384), lambda i: (0, i))],
        out_specs=pl.BlockSpec((3, 16384), lambda i: (0, i)),
        out_shape=jax.ShapeDtypeStruct(xr.shape, xr.dtype),
        compiler_params=pltpu.CompilerParams(
            dimension_semantics=("parallel",),
        ),
    )(xr)
    return out.reshape(x.shape)
